# baseline (speedup 1.0000x reference)
import jax
import jax.numpy as jnp
from jax.experimental import pallas as pl

CB = 4


def _body2d(x_ref, t_ref, o_ref):
    rows = t_ref.shape[0]
    for j in range(x_ref.shape[0] // rows):
        sl = pl.ds(j * rows, rows)
        o_ref[sl, :] = x_ref[sl, :] + t_ref[...]


def _body3d(x_ref, t_ref, o_ref):
    o_ref[...] = x_ref[...] + t_ref[...]


def kernel(encoded_patches, pos_table):
    B, P, D = encoded_patches.shape
    if (P * D) % 128 == 0 and B % CB == 0:
        rows = P * D // 128
        x2d = encoded_patches.reshape(B * rows, 128)
        t2d = pos_table.reshape(rows, 128)
        out2d = pl.pallas_call(
            _body2d,
            grid=(B // CB,),
            in_specs=[
                pl.BlockSpec((CB * rows, 128), lambda i: (i, 0)),
                pl.BlockSpec((rows, 128), lambda i: (0, 0)),
            ],
            out_specs=pl.BlockSpec((CB * rows, 128), lambda i: (i, 0)),
            out_shape=jax.ShapeDtypeStruct((B * rows, 128), jnp.float32),
        )(x2d, t2d)
        return out2d.reshape(B, P, D)
    return pl.pallas_call(
        _body3d,
        grid=(B,),
        in_specs=[
            pl.BlockSpec((1, P, D), lambda i: (i, 0, 0)),
            pl.BlockSpec((P, D), lambda i: (0, 0)),
        ],
        out_specs=pl.BlockSpec((1, P, D), lambda i: (i, 0, 0)),
        out_shape=jax.ShapeDtypeStruct((B, P, D), jnp.float32),
    )(encoded_patches, pos_table)


# speedup vs baseline: 6.7431x; 6.7431x over previous
import jax
import jax.numpy as jnp
from jax.experimental import pallas as pl

CB = 4


def _body(x_ref, t_ref, o_ref):
    o_ref[...] = x_ref[...] + t_ref[...]


def kernel(encoded_patches, pos_table):
    B, P, D = encoded_patches.shape
    xt = jnp.swapaxes(encoded_patches, 1, 2)
    tt = pos_table.T
    cb = CB if B % CB == 0 else 1
    out_t = pl.pallas_call(
        _body,
        grid=(B // cb,),
        in_specs=[
            pl.BlockSpec((cb, D, P), lambda i: (i, 0, 0)),
            pl.BlockSpec((D, P), lambda i: (0, 0)),
        ],
        out_specs=pl.BlockSpec((cb, D, P), lambda i: (i, 0, 0)),
        out_shape=jax.ShapeDtypeStruct((B, D, P), jnp.float32),
    )(xt, tt)
    return jnp.swapaxes(out_t, 1, 2)


# speedup vs baseline: 8.1112x; 1.2029x over previous
import jax
import jax.numpy as jnp
from jax.experimental import pallas as pl

CB = 8


def _body(x_ref, t_ref, o_ref):
    o_ref[...] = x_ref[...] + t_ref[...]


def kernel(encoded_patches, pos_table):
    B, P, D = encoded_patches.shape
    xt = jnp.swapaxes(encoded_patches, 1, 2)
    tt = pos_table.T
    cb = CB if B % CB == 0 else 1
    out_t = pl.pallas_call(
        _body,
        grid=(B // cb,),
        in_specs=[
            pl.BlockSpec((cb, D, P), lambda i: (i, 0, 0)),
            pl.BlockSpec((D, P), lambda i: (0, 0)),
        ],
        out_specs=pl.BlockSpec((cb, D, P), lambda i: (i, 0, 0)),
        out_shape=jax.ShapeDtypeStruct((B, D, P), jnp.float32),
    )(xt, tt)
    return jnp.swapaxes(out_t, 1, 2)


# speedup vs baseline: 8.4235x; 1.0385x over previous
import jax
import jax.numpy as jnp
from jax.experimental import pallas as pl

CB = 16


def _body(x_ref, t_ref, o_ref):
    o_ref[...] = x_ref[...] + t_ref[...]


def kernel(encoded_patches, pos_table):
    B, P, D = encoded_patches.shape
    xt = jnp.swapaxes(encoded_patches, 1, 2)
    tt = pos_table.T
    cb = CB if B % CB == 0 else 1
    out_t = pl.pallas_call(
        _body,
        grid=(B // cb,),
        in_specs=[
            pl.BlockSpec((cb, D, P), lambda i: (i, 0, 0)),
            pl.BlockSpec((D, P), lambda i: (0, 0)),
        ],
        out_specs=pl.BlockSpec((cb, D, P), lambda i: (i, 0, 0)),
        out_shape=jax.ShapeDtypeStruct((B, D, P), jnp.float32),
    )(xt, tt)
    return jnp.swapaxes(out_t, 1, 2)


# speedup vs baseline: 8.6316x; 1.0247x over previous
import jax
import jax.numpy as jnp
from jax.experimental import pallas as pl

CB = 32


def _body(x_ref, t_ref, o_ref):
    o_ref[...] = x_ref[...] + t_ref[...]


def kernel(encoded_patches, pos_table):
    B, P, D = encoded_patches.shape
    xt = jnp.swapaxes(encoded_patches, 1, 2)
    tt = pos_table.T
    cb = CB if B % CB == 0 else 1
    out_t = pl.pallas_call(
        _body,
        grid=(B // cb,),
        in_specs=[
            pl.BlockSpec((cb, D, P), lambda i: (i, 0, 0)),
            pl.BlockSpec((D, P), lambda i: (0, 0)),
        ],
        out_specs=pl.BlockSpec((cb, D, P), lambda i: (i, 0, 0)),
        out_shape=jax.ShapeDtypeStruct((B, D, P), jnp.float32),
    )(xt, tt)
    return jnp.swapaxes(out_t, 1, 2)
